# Initial kernel scaffold; baseline (speedup 1.0000x reference)
import jax
import jax.numpy as jnp
from jax.experimental import pallas as pl


def kernel(x, emb):
    raise NotImplementedError("write your pallas kernel here")


# speedup vs baseline: 1.7201x; 1.7201x over previous
import jax
import jax.numpy as jnp
from jax.experimental import pallas as pl


_TILE_T = 256


def _add_pe_kernel(x_ref, emb_ref, out_ref):
    out_ref[...] = x_ref[...] + emb_ref[...][None, :, :]


def kernel(x, emb):
    b, t, d = x.shape
    grid = (t // _TILE_T,)
    return pl.pallas_call(
        _add_pe_kernel,
        grid=grid,
        in_specs=[
            pl.BlockSpec((b, _TILE_T, d), lambda i: (0, i, 0)),
            pl.BlockSpec((_TILE_T, d), lambda i: (i, 0)),
        ],
        out_specs=pl.BlockSpec((b, _TILE_T, d), lambda i: (0, i, 0)),
        out_shape=jax.ShapeDtypeStruct((b, t, d), x.dtype),
    )(x, emb[:t])


# speedup vs baseline: 1.7227x; 1.0015x over previous
import jax
import jax.numpy as jnp
from jax.experimental import pallas as pl


_TILE_T = 512


def _add_pe_kernel(x_ref, emb_ref, out_ref):
    out_ref[...] = x_ref[...] + emb_ref[...][None, :, :]


def kernel(x, emb):
    b, t, d = x.shape
    grid = (t // _TILE_T,)
    return pl.pallas_call(
        _add_pe_kernel,
        grid=grid,
        in_specs=[
            pl.BlockSpec((b, _TILE_T, d), lambda i: (0, i, 0)),
            pl.BlockSpec((_TILE_T, d), lambda i: (i, 0)),
        ],
        out_specs=pl.BlockSpec((b, _TILE_T, d), lambda i: (0, i, 0)),
        out_shape=jax.ShapeDtypeStruct((b, t, d), x.dtype),
    )(x, emb[:t])
